# baseline (speedup 1.0000x reference)
import functools

import jax
import jax.numpy as jnp
from jax import lax
from jax.experimental import pallas as pl
from jax.experimental.pallas import tpu as pltpu
from jax.experimental.pallas import tpu_sc as plsc

CHUNK = 128
SUPER = 8
N_SUBCORES = 16
N_CORES = 2
N_WORKERS = N_CORES * N_SUBCORES
D = 128


def _sc_aggregate(x, col2d, row2d, ew2d, n_pad, tile_e):
    n_chunks = tile_e // CHUNK
    n_super = n_chunks // SUPER
    rows_per_tile = n_pad // N_SUBCORES
    rp = rows_per_tile // 5
    mesh = plsc.VectorSubcoreMesh(core_axis_name="c", subcore_axis_name="s")

    @functools.partial(
        pl.kernel,
        mesh=mesh,
        out_type=jax.ShapeDtypeStruct((N_CORES, n_pad, D), jnp.float32),
        scratch_types=[
            pltpu.VMEM((CHUNK, D), jnp.float32),
            pltpu.VMEM((CHUNK, D), jnp.float32),
            pltpu.VMEM((SUPER, 128), jnp.int32),
            pltpu.VMEM((SUPER, 128), jnp.int32),
            pltpu.VMEM((SUPER, 128), jnp.int32),
            pltpu.VMEM((SUPER, 128), jnp.int32),
            pltpu.VMEM((SUPER, 128), jnp.float32),
            pltpu.VMEM((SUPER, 128), jnp.float32),
            pltpu.VMEM_SHARED((n_pad, D), jnp.float32),
            pltpu.SemaphoreType.DMA,
            pltpu.SemaphoreType.DMA,
            pltpu.SemaphoreType.DMA,
            pltpu.SemaphoreType.DMA,
            pltpu.SemaphoreType.DMA,
            pltpu.SemaphoreType.DMA,
        ],
    )
    def agg_kernel(x_hbm, col_hbm, row_hbm, ew_hbm, out_hbm,
                   buf0, buf1, colb0, colb1, rowb0, rowb1, ewb0, ewb1,
                   acc, sm0, sm1, sg0, sg1, ss0, ss1):
        cid = lax.axis_index("c")
        sid = lax.axis_index("s")
        wid = cid * N_SUBCORES + sid
        chunk0 = wid * n_chunks
        bsets = [(buf0, sg0, ss0), (buf1, sg1, ss1)]
        msets = [(colb0, rowb0, ewb0, sm0), (colb1, rowb1, ewb1, sm1)]
        zero = jnp.zeros((16,), jnp.float32)

        def md_start(sc, m):
            base = chunk0 + sc * SUPER
            pltpu.async_copy(col_hbm.at[pl.ds(base, SUPER)], m[0], m[3])
            pltpu.async_copy(row_hbm.at[pl.ds(base, SUPER)], m[1], m[3])
            pltpu.async_copy(ew_hbm.at[pl.ds(base, SUPER)], m[2], m[3])

        def md_wait(m):
            for i in range(3):
                pltpu.make_async_copy(col_hbm.at[pl.ds(0, SUPER)],
                                      m[i], m[3]).wait()

        def gather_start(m, j, s):
            pltpu.async_copy(x_hbm.at[m[0].at[j]], s[0], s[1])

        def gather_wait(s):
            pltpu.make_async_copy(x_hbm.at[colb0.at[0]], s[0], s[1]).wait()

        def scale(m, j, s):
            def scale16(it, c2):
                e0 = it * 16
                wv = m[2][j, pl.ds(e0, 16)]
                for u in range(16):
                    w = wv[u]
                    for v in range(D // 16):
                        s[0][e0 + u, pl.ds(v * 16, 16)] = (
                            s[0][e0 + u, pl.ds(v * 16, 16)] * w)
                return c2
            lax.fori_loop(0, CHUNK // 16, scale16, 0)

        def scatter_start(m, j, s):
            pltpu.async_copy(s[0], acc.at[m[1].at[j]], s[2], add=True)

        def scatter_wait(s):
            pltpu.make_async_copy(s[0], acc.at[rowb0.at[0]], s[2]).wait()

        md_start(0, msets[0])

        def zrow(r, carry):
            for v in range(D // 16):
                buf0[r, pl.ds(v * 16, 16)] = zero
            return carry

        lax.fori_loop(0, rp, zrow, 0)
        for i in range(5):
            pltpu.sync_copy(buf0.at[pl.ds(0, rp)],
                            acc.at[pl.ds(sid * rows_per_tile + i * rp, rp)])
        plsc.subcore_barrier()
        md_wait(msets[0])
        gather_start(msets[0], 0, bsets[0])

        def outer(g_, carry):
            for bb in (0, 1):
                sc = 2 * g_ + bb
                m, nm = msets[bb], msets[1 - bb]
                for j in range(SUPER):
                    s, ns = bsets[j % 2], bsets[1 - j % 2]

                    def stage_next(nj_m, nj):
                        if bb == 0 and j == 0:
                            @pl.when(g_ > 0)
                            def _():
                                scatter_wait(ns)
                        else:
                            scatter_wait(ns)
                        gather_start(nj_m, nj, ns)

                    gather_wait(s)
                    scale(m, j, s)
                    if j < SUPER - 1:
                        stage_next(m, j + 1)
                    elif bb == 0:
                        stage_next(nm, 0)
                    else:
                        @pl.when(g_ < n_super // 2 - 1)
                        def _():
                            stage_next(nm, 0)
                    scatter_start(m, j, s)
                    if j == 0:
                        if bb == 0:
                            md_start(sc + 1, nm)
                        else:
                            @pl.when(g_ < n_super // 2 - 1)
                            def _():
                                md_start(sc + 1, nm)
                    if j == SUPER - 2:
                        if bb == 0:
                            md_wait(nm)
                        else:
                            @pl.when(g_ < n_super // 2 - 1)
                            def _():
                                md_wait(nm)
            return carry

        lax.fori_loop(0, n_super // 2, outer, 0)
        for b in (0, 1):
            scatter_wait(bsets[b])
        plsc.subcore_barrier()
        for i in range(5):
            pltpu.sync_copy(acc.at[pl.ds(sid * rows_per_tile + i * rp, rp)],
                            out_hbm.at[cid].at[pl.ds(sid * rows_per_tile + i * rp, rp)])

    return agg_kernel(x, col2d, row2d, ew2d)


def _tc_matmul(agg, weight, n_nodes):
    bn = n_nodes // 5

    def mm(aref, wref, oref):
        oref[...] = jnp.dot(aref[0] + aref[1], wref[...],
                            preferred_element_type=jnp.float32)

    return pl.pallas_call(
        mm,
        grid=(5,),
        in_specs=[
            pl.BlockSpec((N_CORES, bn, D), lambda i: (0, i, 0)),
            pl.BlockSpec(weight.shape, lambda i: (0, 0)),
        ],
        out_specs=pl.BlockSpec((bn, weight.shape[1]), lambda i: (i, 0)),
        out_shape=jax.ShapeDtypeStruct((n_nodes, weight.shape[1]),
                                       jnp.float32),
    )(agg, weight)


def kernel(x, edge_index, edge_weight, weight):
    n_nodes = x.shape[0]
    n_edges = edge_weight.shape[0]
    unit = N_WORKERS * 2 * SUPER * CHUNK
    e_pad = -(-n_edges // unit) * unit
    per_tile = e_pad // N_WORKERS
    npad = e_pad - n_edges

    ei = edge_index.astype(jnp.int32)
    pad_ids = (jnp.arange(npad, dtype=jnp.int32) * 97) % n_nodes
    gcol = jnp.concatenate([ei[1], pad_ids])
    grow = jnp.concatenate([ei[0], pad_ids])
    ew = jnp.concatenate([edge_weight.astype(jnp.float32),
                          jnp.zeros((npad,), jnp.float32)])
    col2d = gcol.reshape(e_pad // 128, 128)
    row2d = grow.reshape(e_pad // 128, 128)
    ew2d = ew.reshape(e_pad // 128, 128)

    n_pad = -(-n_nodes // (8 * N_SUBCORES * 5)) * (8 * N_SUBCORES * 5)
    agg = _sc_aggregate(x, col2d, row2d, ew2d, n_pad, per_tile)
    return _tc_matmul(agg, weight, n_nodes)


# speedup vs baseline: 1.2710x; 1.2710x over previous
import functools

import jax
import jax.numpy as jnp
from jax import lax
from jax.experimental import pallas as pl
from jax.experimental.pallas import tpu as pltpu
from jax.experimental.pallas import tpu_sc as plsc

CHUNK = 128
SUPER = 8
N_SUBCORES = 16
N_CORES = 2
N_WORKERS = N_CORES * N_SUBCORES
D = 128


def _sc_aggregate(x, col2d, row2d, ew2d, n_pad, tile_e):
    n_chunks = tile_e // CHUNK
    n_super = n_chunks // SUPER
    rows_per_tile = n_pad // N_SUBCORES
    rp = rows_per_tile // 5
    mesh = plsc.VectorSubcoreMesh(core_axis_name="c", subcore_axis_name="s")

    @functools.partial(
        pl.kernel,
        mesh=mesh,
        out_type=jax.ShapeDtypeStruct((N_CORES, n_pad, D), jnp.float32),
        scratch_types=[
            pltpu.VMEM((CHUNK, D), jnp.float32),
            pltpu.VMEM((CHUNK, D), jnp.float32),
            pltpu.VMEM((SUPER, 128), jnp.int32),
            pltpu.VMEM((SUPER, 128), jnp.int32),
            pltpu.VMEM((SUPER, 128), jnp.int32),
            pltpu.VMEM((SUPER, 128), jnp.int32),
            pltpu.VMEM((SUPER, 128), jnp.float32),
            pltpu.VMEM((SUPER, 128), jnp.float32),
            pltpu.VMEM_SHARED((n_pad, D), jnp.float32),
            pltpu.SemaphoreType.DMA,
            pltpu.SemaphoreType.DMA,
            pltpu.SemaphoreType.DMA,
            pltpu.SemaphoreType.DMA,
            pltpu.SemaphoreType.DMA,
            pltpu.SemaphoreType.DMA,
        ],
    )
    def agg_kernel(x_hbm, col_hbm, row_hbm, ew_hbm, out_hbm,
                   buf0, buf1, colb0, colb1, rowb0, rowb1, ewb0, ewb1,
                   acc, sm0, sm1, sg0, sg1, ss0, ss1):
        cid = lax.axis_index("c")
        sid = lax.axis_index("s")
        wid = cid * N_SUBCORES + sid
        chunk0 = wid * n_chunks
        bsets = [(buf0, sg0, ss0), (buf1, sg1, ss1)]
        msets = [(colb0, rowb0, ewb0, sm0), (colb1, rowb1, ewb1, sm1)]
        zero = jnp.zeros((16,), jnp.float32)

        def md_start(sc, m):
            base = chunk0 + sc * SUPER
            pltpu.async_copy(col_hbm.at[pl.ds(base, SUPER)], m[0], m[3])
            pltpu.async_copy(row_hbm.at[pl.ds(base, SUPER)], m[1], m[3])
            pltpu.async_copy(ew_hbm.at[pl.ds(base, SUPER)], m[2], m[3])

        def md_wait(m):
            for i in range(3):
                pltpu.make_async_copy(col_hbm.at[pl.ds(0, SUPER)],
                                      m[i], m[3]).wait()

        def gather_start(m, j, s):
            pltpu.async_copy(x_hbm.at[m[0].at[j]], s[0], s[1])

        def gather_wait(s):
            pltpu.make_async_copy(x_hbm.at[colb0.at[0]], s[0], s[1]).wait()

        def scale(m, j, s):
            def scale16(it, c2):
                e0 = it * 16
                wv = m[2][j, pl.ds(e0, 16)]
                for u in range(16):
                    w = wv[u]
                    for v in range(D // 16):
                        s[0][e0 + u, pl.ds(v * 16, 16)] = (
                            s[0][e0 + u, pl.ds(v * 16, 16)] * w)
                return c2
            lax.fori_loop(0, CHUNK // 16, scale16, 0)

        def scatter_start(m, j, s):
            pltpu.async_copy(s[0], acc.at[m[1].at[j]], s[2], add=True)

        def scatter_wait(s):
            pltpu.make_async_copy(s[0], acc.at[rowb0.at[0]], s[2]).wait()

        md_start(0, msets[0])

        def zrow(r, carry):
            for v in range(D // 16):
                buf0[r, pl.ds(v * 16, 16)] = zero
            return carry

        lax.fori_loop(0, rp, zrow, 0)
        for i in range(5):
            pltpu.sync_copy(buf0.at[pl.ds(0, rp)],
                            acc.at[pl.ds(sid * rows_per_tile + i * rp, rp)])
        plsc.subcore_barrier()
        md_wait(msets[0])
        gather_start(msets[0], 0, bsets[0])

        def outer(g_, carry):
            for bb in (0, 1):
                sc = 2 * g_ + bb
                m, nm = msets[bb], msets[1 - bb]
                for j in range(SUPER):
                    s, ns = bsets[j % 2], bsets[1 - j % 2]

                    def stage_next(nj_m, nj):
                        if bb == 0 and j == 0:
                            @pl.when(g_ > 0)
                            def _():
                                scatter_wait(ns)
                        else:
                            scatter_wait(ns)
                        gather_start(nj_m, nj, ns)

                    if j < SUPER - 1:
                        stage_next(m, j + 1)
                    elif bb == 0:
                        stage_next(nm, 0)
                    else:
                        @pl.when(g_ < n_super // 2 - 1)
                        def _():
                            stage_next(nm, 0)
                    gather_wait(s)
                    scale(m, j, s)
                    scatter_start(m, j, s)
                    if j == 0:
                        if bb == 0:
                            md_start(sc + 1, nm)
                        else:
                            @pl.when(g_ < n_super // 2 - 1)
                            def _():
                                md_start(sc + 1, nm)
                    if j == SUPER - 2:
                        if bb == 0:
                            md_wait(nm)
                        else:
                            @pl.when(g_ < n_super // 2 - 1)
                            def _():
                                md_wait(nm)
            return carry

        lax.fori_loop(0, n_super // 2, outer, 0)
        for b in (0, 1):
            scatter_wait(bsets[b])
        plsc.subcore_barrier()
        for i in range(5):
            pltpu.sync_copy(acc.at[pl.ds(sid * rows_per_tile + i * rp, rp)],
                            out_hbm.at[cid].at[pl.ds(sid * rows_per_tile + i * rp, rp)])

    return agg_kernel(x, col2d, row2d, ew2d)


def _tc_matmul(agg, weight, n_nodes):
    bn = n_nodes // 5

    def mm(aref, wref, oref):
        oref[...] = jnp.dot(aref[0] + aref[1], wref[...],
                            preferred_element_type=jnp.float32)

    return pl.pallas_call(
        mm,
        grid=(5,),
        in_specs=[
            pl.BlockSpec((N_CORES, bn, D), lambda i: (0, i, 0)),
            pl.BlockSpec(weight.shape, lambda i: (0, 0)),
        ],
        out_specs=pl.BlockSpec((bn, weight.shape[1]), lambda i: (i, 0)),
        out_shape=jax.ShapeDtypeStruct((n_nodes, weight.shape[1]),
                                       jnp.float32),
    )(agg, weight)


def kernel(x, edge_index, edge_weight, weight):
    n_nodes = x.shape[0]
    n_edges = edge_weight.shape[0]
    unit = N_WORKERS * 2 * SUPER * CHUNK
    e_pad = -(-n_edges // unit) * unit
    per_tile = e_pad // N_WORKERS
    npad = e_pad - n_edges

    ei = edge_index.astype(jnp.int32)
    pad_ids = (jnp.arange(npad, dtype=jnp.int32) * 97) % n_nodes
    gcol = jnp.concatenate([ei[1], pad_ids])
    grow = jnp.concatenate([ei[0], pad_ids])
    ew = jnp.concatenate([edge_weight.astype(jnp.float32),
                          jnp.zeros((npad,), jnp.float32)])
    col2d = gcol.reshape(e_pad // 128, 128)
    row2d = grow.reshape(e_pad // 128, 128)
    ew2d = ew.reshape(e_pad // 128, 128)

    n_pad = -(-n_nodes // (8 * N_SUBCORES * 5)) * (8 * N_SUBCORES * 5)
    agg = _sc_aggregate(x, col2d, row2d, ew2d, n_pad, per_tile)
    return _tc_matmul(agg, weight, n_nodes)
